# Initial kernel scaffold; baseline (speedup 1.0000x reference)
import jax
import jax.numpy as jnp
from jax.experimental import pallas as pl


def kernel(x, tables):
    raise NotImplementedError("write your pallas kernel here")


# speedup vs baseline: 2.4106x; 2.4106x over previous
import functools

import jax
import jax.numpy as jnp
from jax import lax
from jax.experimental import pallas as pl
from jax.experimental.pallas import tpu as pltpu
from jax.experimental.pallas import tpu_sc as plsc

F = 26
D = 16
V = 100000
PAIRS = [(i, j) for i in range(F) for j in range(i + 1, F)]
NPAIR = len(PAIRS)
NOUT = NPAIR + F


@functools.lru_cache(maxsize=None)
def _sc_call(batch):
    info = plsc.get_sparse_core_info()
    nw = info.num_cores * info.num_subcores
    assert batch % nw == 0
    per_w = batch // nw
    mesh = plsc.VectorSubcoreMesh(core_axis_name="c", subcore_axis_name="s")

    @functools.partial(
        pl.kernel,
        out_type=jax.ShapeDtypeStruct((batch, NOUT, D), jnp.float32),
        mesh=mesh,
        scratch_types=[
            pltpu.VMEM((F, F), jnp.int32),
            pltpu.VMEM((F, F, D), jnp.float32),
            pltpu.VMEM((NOUT, D), jnp.float32),
            pltpu.SemaphoreType.DMA,
        ],
        compiler_params=pltpu.CompilerParams(use_tc_tiling_on_sc=False),
    )
    def k(tab_hbm, idx_hbm, out_hbm, idx_v, rows_v, outb_v, sem):
        wid = lax.axis_index("s") * info.num_cores + lax.axis_index("c")
        base = wid * per_w

        def body(e, carry):
            b = base + e
            pltpu.sync_copy(idx_hbm.at[b], idx_v)
            copies = [
                pltpu.async_copy(tab_hbm.at[idx_v.at[s]], rows_v.at[s], sem)
                for s in range(F)
            ]
            for c in copies:
                c.wait()
            for p, (i, j) in enumerate(PAIRS):
                outb_v[p] = rows_v[i, j] * rows_v[j, i]
            for i in range(F):
                outb_v[NPAIR + i] = rows_v[i, i]
            pltpu.sync_copy(outb_v, out_hbm.at[b])
            return carry

        lax.fori_loop(0, per_w, body, 0)

    return k


def kernel(x, tables):
    batch = x.shape[0]
    flat_tables = tables.reshape(F * V, D)
    offs = (jnp.arange(F, dtype=jnp.int32) * V)[None, None, :]
    idx = x.astype(jnp.int32)[:, :, None] + offs
    out = _sc_call(batch)(flat_tables, idx)
    return out.reshape(batch, NOUT * D)


# speedup vs baseline: 3.2601x; 1.3524x over previous
import functools

import jax
import jax.numpy as jnp
from jax import lax
from jax.experimental import pallas as pl
from jax.experimental.pallas import tpu as pltpu
from jax.experimental.pallas import tpu_sc as plsc

F = 26
FP = 32
D = 16
V = 100000
NC = 4
NI = F * NC
PAIRS = [(i, j) for i in range(F) for j in range(i + 1, F)]
NPAIR = len(PAIRS)
NCOL = (NPAIR + F) * D


def _chunk(s, t):
    return s * NC + t // 8, (t % 8) * D


@functools.lru_cache(maxsize=None)
def _sc_call(batch):
    info = plsc.get_sparse_core_info()
    nw = info.num_cores * info.num_subcores
    assert batch % nw == 0
    per_w = batch // nw
    mesh = plsc.VectorSubcoreMesh(core_axis_name="c", subcore_axis_name="s")

    @functools.partial(
        pl.kernel,
        out_type=jax.ShapeDtypeStruct((batch, NCOL), jnp.float32),
        mesh=mesh,
        scratch_types=[
            pltpu.VMEM((per_w, NI), jnp.int32),
            pltpu.VMEM((NI, 128), jnp.float32),
            pltpu.VMEM((NCOL,), jnp.float32),
            pltpu.SemaphoreType.DMA,
        ],
    )
    def k(tabt_hbm, xq_hbm, out_hbm, xv, rows_v, outb_v, gsem):
        wid = lax.axis_index("s") * info.num_cores + lax.axis_index("c")
        base = wid * per_w
        pltpu.sync_copy(xq_hbm.at[pl.ds(base, per_w)], xv)

        def body(e, carry):
            b = base + e
            pltpu.async_copy(tabt_hbm.at[xv.at[e]], rows_v, gsem).wait()
            for p, (i, j) in enumerate(PAIRS):
                ra, ca = _chunk(i, j)
                rb, cb = _chunk(j, i)
                outb_v[pl.ds(p * D, D)] = (
                    rows_v[ra, pl.ds(ca, D)] * rows_v[rb, pl.ds(cb, D)]
                )
            for i in range(F):
                r, c = _chunk(i, i)
                outb_v[pl.ds((NPAIR + i) * D, D)] = rows_v[r, pl.ds(c, D)]
            pltpu.sync_copy(outb_v, out_hbm.at[b])
            return carry

        lax.fori_loop(0, per_w, body, 0)

    return k


def kernel(x, tables):
    batch = x.shape[0]
    pad = jnp.zeros((FP - F, V, D), jnp.float32)
    tabt = jnp.concatenate([tables, pad], axis=0).transpose(1, 0, 2)
    tabt = tabt.reshape(V * NC, 128)
    xq = (x.astype(jnp.int32)[:, :, None] * NC
          + jnp.arange(NC, dtype=jnp.int32)[None, None, :]).reshape(batch, NI)
    return _sc_call(batch)(tabt, xq)
